# Initial kernel scaffold; baseline (speedup 1.0000x reference)
import jax
import jax.numpy as jnp
from jax.experimental import pallas as pl


def kernel(hidden_states, router_w, w1, w2):
    raise NotImplementedError("write your pallas kernel here")


# speedup vs baseline: 1.3565x; 1.3565x over previous
import jax
import jax.numpy as jnp
from jax.experimental import pallas as pl
from jax.experimental.pallas import tpu as pltpu

E = 8
TOP_K = 2
NEG_INF = -1e30


def _moe_dense_kernel(x_ref, rw_ref, w1_ref, w2_ref, out_ref, acc_ref):
    e = pl.program_id(0)
    t = pl.program_id(1)
    bt = x_ref.shape[0]
    x = x_ref[...]
    logits = jax.lax.dot_general(
        x, rw_ref[...], (((1,), (0,)), ((), ())),
        preferred_element_type=jnp.float32)
    eids = jax.lax.broadcasted_iota(jnp.int32, logits.shape, 1)
    m1 = jnp.max(logits, axis=-1, keepdims=True)
    cand1 = jnp.where(logits >= m1, eids, E)
    e1 = jnp.min(cand1, axis=-1, keepdims=True)
    logits2 = jnp.where(eids == e1, NEG_INF, logits)
    m2 = jnp.max(logits2, axis=-1, keepdims=True)
    cand2 = jnp.where(logits2 >= m2, eids, E)
    e2 = jnp.min(cand2, axis=-1, keepdims=True)
    b = jnp.exp(m2 - m1)
    g1 = 1.0 / (1.0 + b)
    g2 = b / (1.0 + b)
    gate = jnp.where(e1 == e, g1, 0.0) + jnp.where(e2 == e, g2, 0.0)

    h = jax.lax.dot_general(
        x, w1_ref[0], (((1,), (0,)), ((), ())),
        preferred_element_type=jnp.float32)
    h = jnp.maximum(h, 0.0)
    y = jax.lax.dot_general(
        h, w2_ref[0], (((1,), (0,)), ((), ())),
        preferred_element_type=jnp.float32)

    @pl.when(e == 0)
    def _():
        acc_ref[pl.ds(t * bt, bt), :] = gate * y

    @pl.when(e > 0)
    def _():
        acc_ref[pl.ds(t * bt, bt), :] += gate * y

    @pl.when(e == E - 1)
    def _():
        out_ref[...] = acc_ref[pl.ds(t * bt, bt), :]


def kernel(hidden_states, router_w, w1, w2):
    b, s, d = hidden_states.shape
    T = b * s
    d_ff = w1.shape[-1]
    flat = hidden_states.reshape(T, d)
    BT = 512
    grid = (E, T // BT)
    out = pl.pallas_call(
        _moe_dense_kernel,
        grid=grid,
        in_specs=[
            pl.BlockSpec((BT, d), lambda e, t: (t, 0)),
            pl.BlockSpec((d, E), lambda e, t: (0, 0)),
            pl.BlockSpec((1, d, d_ff), lambda e, t: (e, 0, 0)),
            pl.BlockSpec((1, d_ff, d), lambda e, t: (e, 0, 0)),
        ],
        out_specs=pl.BlockSpec((BT, d), lambda e, t: (t, 0)),
        out_shape=jax.ShapeDtypeStruct((T, d), jnp.float32),
        scratch_shapes=[pltpu.VMEM((T, d), jnp.float32)],
        compiler_params=pltpu.CompilerParams(
            dimension_semantics=("arbitrary", "arbitrary"),
        ),
    )(flat, router_w, w1, w2)
    return out.reshape(b, s, d)


# speedup vs baseline: 1.5187x; 1.1196x over previous
import functools

import jax
import jax.numpy as jnp
from jax import lax
from jax.experimental import pallas as pl
from jax.experimental.pallas import tpu as pltpu
from jax.experimental.pallas import tpu_sc as plsc

E = 8
TOP_K = 2
D_MODEL = 1024
D_FF = 2048
T = 2048
BLK = 256
NB = 24
P = NB * BLK
NEG_INF = -1e30

NC = 2
NS = 16
NW = NC * NS
TPW = T // NW
CH = 32

_HI = jax.lax.Precision.HIGHEST


def _route_kernel(x_ref, rw_ref, d0_ref, d1_ref, g0_ref, g1_ref, eid_ref):
    x = x_ref[...]
    logits = lax.dot_general(x, rw_ref[...], (((1,), (0,)), ((), ())),
                             preferred_element_type=jnp.float32)
    eids = lax.broadcasted_iota(jnp.int32, (T, E), 1)
    m1 = jnp.max(logits, axis=-1, keepdims=True)
    e1 = jnp.min(jnp.where(logits >= m1, eids, E), axis=-1, keepdims=True)
    l2 = jnp.where(eids == e1, NEG_INF, logits)
    m2 = jnp.max(l2, axis=-1, keepdims=True)
    e2 = jnp.min(jnp.where(l2 >= m2, eids, E), axis=-1, keepdims=True)
    r = jnp.exp(m2 - m1)
    g0_ref[...] = 1.0 / (1.0 + r)
    g1_ref[...] = r / (1.0 + r)

    oh0 = (eids == e1).astype(jnp.float32)
    oh1 = (eids == e2).astype(jnp.float32)
    tri = (lax.broadcasted_iota(jnp.int32, (128, 128), 0)
           >= lax.broadcasted_iota(jnp.int32, (128, 128), 1)).astype(jnp.float32)
    parts0, parts1 = [], []
    carry0 = jnp.zeros((1, E), jnp.float32)
    carry1 = jnp.zeros((1, E), jnp.float32)
    for k in range(T // 128):
        p0 = lax.dot_general(tri, oh0[k * 128:(k + 1) * 128], (((1,), (0,)), ((), ())),
                             precision=_HI, preferred_element_type=jnp.float32) + carry0
        p1 = lax.dot_general(tri, oh1[k * 128:(k + 1) * 128], (((1,), (0,)), ((), ())),
                             precision=_HI, preferred_element_type=jnp.float32) + carry1
        parts0.append(p0)
        parts1.append(p1)
        carry0 = p0[-1:, :]
        carry1 = p1[-1:, :]
    c0 = jnp.concatenate(parts0, axis=0)
    c1 = jnp.concatenate(parts1, axis=0)
    c0ex = c0 - oh0
    c1ex = c1 - oh1

    cnt = carry0 + carry1
    pad_cnt = (((cnt.astype(jnp.int32) + (BLK - 1)) >> 8) << 8).astype(jnp.float32)
    m8 = (lax.broadcasted_iota(jnp.int32, (E, E), 0)
          < lax.broadcasted_iota(jnp.int32, (E, E), 1)).astype(jnp.float32)
    off = lax.dot_general(pad_cnt, m8, (((1,), (0,)), ((), ())),
                          precision=_HI, preferred_element_type=jnp.float32)

    rank0 = c0ex + c1ex
    rank1 = c0 + c1ex
    d0_ref[...] = jnp.sum(oh0 * (off + rank0), axis=-1, keepdims=True).astype(jnp.int32)
    d1_ref[...] = jnp.sum(oh1 * (off + rank1), axis=-1, keepdims=True).astype(jnp.int32)

    pend = (off + pad_cnt).astype(jnp.int32)
    bstart = lax.broadcasted_iota(jnp.int32, (NB, E), 0) * BLK
    n_before = jnp.sum((pend <= bstart).astype(jnp.int32), axis=-1, keepdims=True)
    eid_ref[...] = jnp.minimum(n_before, E - 1)


def _ffn_kernel(eid_ref, x_ref, w1_ref, w2_ref, y_ref):
    del eid_ref
    h = lax.dot_general(x_ref[...], w1_ref[0], (((1,), (0,)), ((), ())),
                        preferred_element_type=jnp.float32)
    h = jnp.maximum(h, 0.0)
    y_ref[...] = lax.dot_general(h, w2_ref[0], (((1,), (0,)), ((), ())),
                                 preferred_element_type=jnp.float32)


def _combine_kernel(y0_ref, y1_ref, g0_ref, g1_ref, out_ref):
    out_ref[...] = g0_ref[...] * y0_ref[...] + g1_ref[...] * y1_ref[...]


def _sc_mesh():
    return plsc.VectorSubcoreMesh(core_axis_name="c", subcore_axis_name="s",
                                  num_cores=NC, num_subcores=NS)


def _dispatch_body(flat_hbm, d0_hbm, d1_hbm, xs_hbm, x_v, i0_v, i1_v, s0, s1):
    wid = lax.axis_index("s") * NC + lax.axis_index("c")
    base = wid * TPW
    pltpu.sync_copy(flat_hbm.at[pl.ds(base, TPW)], x_v)
    pltpu.sync_copy(d0_hbm.at[pl.ds(base, TPW)], i0_v)
    pltpu.sync_copy(d1_hbm.at[pl.ds(base, TPW)], i1_v)
    cp0 = pltpu.async_copy(x_v, xs_hbm.at[i0_v], s0)
    cp1 = pltpu.async_copy(x_v, xs_hbm.at[i1_v], s1)
    cp0.wait()
    cp1.wait()


def _sc_dispatch(flat, d0, d1):
    k = pl.kernel(
        _dispatch_body,
        out_type=jax.ShapeDtypeStruct((P, D_MODEL), jnp.float32),
        mesh=_sc_mesh(),
        scratch_types=[
            pltpu.VMEM((TPW, D_MODEL), jnp.float32),
            pltpu.VMEM((TPW,), jnp.int32),
            pltpu.VMEM((TPW,), jnp.int32),
            pltpu.SemaphoreType.DMA,
            pltpu.SemaphoreType.DMA,
        ],
    )
    return k(flat, d0, d1)


def _combine_body(ys_hbm, d0_hbm, d1_hbm, y0_hbm, y1_hbm, rows_v, idx_v, sem):
    wid = lax.axis_index("s") * NC + lax.axis_index("c")
    base = wid * TPW
    for c in range(TPW // CH):
        off = base + c * CH
        pltpu.sync_copy(d0_hbm.at[pl.ds(off, CH)], idx_v)
        pltpu.async_copy(ys_hbm.at[idx_v], rows_v, sem).wait()
        pltpu.sync_copy(rows_v, y0_hbm.at[pl.ds(off, CH)])
        pltpu.sync_copy(d1_hbm.at[pl.ds(off, CH)], idx_v)
        pltpu.async_copy(ys_hbm.at[idx_v], rows_v, sem).wait()
        pltpu.sync_copy(rows_v, y1_hbm.at[pl.ds(off, CH)])


def _sc_combine(ys, d0, d1):
    k = pl.kernel(
        _combine_body,
        out_type=(jax.ShapeDtypeStruct((T, D_MODEL), jnp.float32),
                  jax.ShapeDtypeStruct((T, D_MODEL), jnp.float32)),
        mesh=_sc_mesh(),
        scratch_types=[
            pltpu.VMEM((CH, D_MODEL), jnp.float32),
            pltpu.VMEM((CH,), jnp.int32),
            pltpu.SemaphoreType.DMA,
        ],
    )
    return k(ys, d0, d1)


def kernel(hidden_states, router_w, w1, w2):
    b, s, d = hidden_states.shape
    flat = hidden_states.reshape(T, d)

    d0, d1, g0, g1, eid = pl.pallas_call(
        _route_kernel,
        in_specs=[
            pl.BlockSpec((T, d), lambda: (0, 0)),
            pl.BlockSpec((d, E), lambda: (0, 0)),
        ],
        out_specs=[
            pl.BlockSpec((T, 1), lambda: (0, 0)),
            pl.BlockSpec((T, 1), lambda: (0, 0)),
            pl.BlockSpec((T, 1), lambda: (0, 0)),
            pl.BlockSpec((T, 1), lambda: (0, 0)),
            pl.BlockSpec((NB, 1), lambda: (0, 0)),
        ],
        out_shape=[
            jax.ShapeDtypeStruct((T, 1), jnp.int32),
            jax.ShapeDtypeStruct((T, 1), jnp.int32),
            jax.ShapeDtypeStruct((T, 1), jnp.float32),
            jax.ShapeDtypeStruct((T, 1), jnp.float32),
            jax.ShapeDtypeStruct((NB, 1), jnp.int32),
        ],
    )(flat, router_w)

    d0f = d0.reshape(T)
    d1f = d1.reshape(T)
    eidf = eid.reshape(NB)

    xs = _sc_dispatch(flat, d0f, d1f)

    grid_spec = pltpu.PrefetchScalarGridSpec(
        num_scalar_prefetch=1,
        grid=(NB,),
        in_specs=[
            pl.BlockSpec((BLK, d), lambda i, eid_ref: (i, 0)),
            pl.BlockSpec((1, d, D_FF), lambda i, eid_ref: (eid_ref[i], 0, 0)),
            pl.BlockSpec((1, D_FF, d), lambda i, eid_ref: (eid_ref[i], 0, 0)),
        ],
        out_specs=pl.BlockSpec((BLK, d), lambda i, eid_ref: (i, 0)),
    )
    ys = pl.pallas_call(
        _ffn_kernel,
        grid_spec=grid_spec,
        out_shape=jax.ShapeDtypeStruct((P, d), jnp.float32),
        compiler_params=pltpu.CompilerParams(
            dimension_semantics=("arbitrary",),
        ),
    )(eidf, xs, w1, w2)

    y0, y1 = _sc_combine(ys, d0f, d1f)

    BT = 512
    out = pl.pallas_call(
        _combine_kernel,
        grid=(T // BT,),
        in_specs=[
            pl.BlockSpec((BT, d), lambda t: (t, 0)),
            pl.BlockSpec((BT, d), lambda t: (t, 0)),
            pl.BlockSpec((BT, 1), lambda t: (t, 0)),
            pl.BlockSpec((BT, 1), lambda t: (t, 0)),
        ],
        out_specs=pl.BlockSpec((BT, d), lambda t: (t, 0)),
        out_shape=jax.ShapeDtypeStruct((T, d), jnp.float32),
    )(y0, y1, g0, g1)

    return out.reshape(b, s, d)
